# Initial kernel scaffold; baseline (speedup 1.0000x reference)
import jax
import jax.numpy as jnp
from jax.experimental import pallas as pl


def kernel(node_emb, edge_index, W1, b1, W2, b2):
    raise NotImplementedError("write your pallas kernel here")


# speedup vs baseline: 1.7494x; 1.7494x over previous
import functools

import jax
import jax.numpy as jnp
from jax import lax
from jax.experimental import pallas as pl
from jax.experimental.pallas import tpu as pltpu
from jax.experimental.pallas import tpu_sc as plsc

EMB = 128
HID = 64
NC = 2
NS = 16
NW = NC * NS
L = 16
CHUNK = 128


def _precompute_tables(node_emb, wcat, bcat):
    n = node_emb.shape[0]
    blk = 1000

    def body(emb_ref, w_ref, b_ref, t_ref):
        t_ref[...] = (
            jnp.dot(emb_ref[...], w_ref[...],
                    preferred_element_type=jnp.float32)
            + b_ref[...]
        )

    return pl.pallas_call(
        body,
        grid=(n // blk,),
        in_specs=[
            pl.BlockSpec((blk, EMB), lambda i: (i, 0)),
            pl.BlockSpec((EMB, 2 * HID), lambda i: (0, 0)),
            pl.BlockSpec((1, 2 * HID), lambda i: (0, 0)),
        ],
        out_specs=pl.BlockSpec((blk, 2 * HID), lambda i: (i, 0)),
        out_shape=jax.ShapeDtypeStruct((n, 2 * HID), jnp.float32),
    )(node_emb, wcat, bcat)


def _make_sc_kernel(e_pad):
    ew = e_pad // NW
    nch = ew // CHUNK
    assert nch % 2 == 0

    mesh = plsc.VectorSubcoreMesh(
        core_axis_name="c", subcore_axis_name="s",
        num_cores=NC, num_subcores=NS,
    )

    @functools.partial(
        pl.kernel,
        out_type=jax.ShapeDtypeStruct((e_pad,), jnp.float32),
        mesh=mesh,
        compiler_params=pltpu.CompilerParams(needs_layout_passes=False),
        scratch_types=[
            pltpu.VMEM((ew,), jnp.int32),
            pltpu.VMEM((ew,), jnp.int32),
            pltpu.VMEM((ew,), jnp.float32),
            pltpu.VMEM((ew,), jnp.float32),
            pltpu.VMEM((CHUNK, 2 * HID), jnp.float32),
            pltpu.VMEM((CHUNK, 2 * HID), jnp.float32),
            pltpu.VMEM((CHUNK, 2 * HID), jnp.float32),
            pltpu.VMEM((CHUNK, 2 * HID), jnp.float32),
            pltpu.VMEM((HID, L), jnp.float32),
            pltpu.SemaphoreType.DMA,
            pltpu.SemaphoreType.DMA,
        ],
    )
    def sc_kernel(t_hbm, src_hbm, dst_hbm, noise_hbm, w2b_hbm,
                  out_hbm, idx_s, idx_d, noise_v, out_v,
                  a0, a1, bb0, bb1, w2v, sem0, sem1):
        wid = lax.axis_index("s") * NC + lax.axis_index("c")
        base = wid * ew
        pltpu.sync_copy(src_hbm.at[pl.ds(base, ew)], idx_s)
        pltpu.sync_copy(dst_hbm.at[pl.ds(base, ew)], idx_d)
        pltpu.sync_copy(noise_hbm.at[pl.ds(base, ew)], noise_v)
        pltpu.sync_copy(w2b_hbm, w2v)

        abufs = (a0, a1)
        bbufs = (bb0, bb1)
        sems = (sem0, sem1)

        def fire(c, p):
            pltpu.async_copy(
                t_hbm.at[idx_s.at[pl.ds(c * CHUNK, CHUNK)]], abufs[p], sems[p])
            pltpu.async_copy(
                t_hbm.at[idx_d.at[pl.ds(c * CHUNK, CHUNK)]], bbufs[p], sems[p])

        def wait(c, p):
            pltpu.make_async_copy(
                t_hbm.at[idx_s.at[pl.ds(c * CHUNK, CHUNK)]], abufs[p], sems[p]
            ).wait()
            pltpu.make_async_copy(
                t_hbm.at[idx_d.at[pl.ds(c * CHUNK, CHUNK)]], bbufs[p], sems[p]
            ).wait()

        def compute(c, abuf, bbuf):
            erows = [
                lax.iota(jnp.int32, L) + e0 for e0 in range(0, CHUNK, L)
            ]
            accs = [jnp.zeros((L,), jnp.float32) for _ in range(CHUNK // L)]
            for j in range(HID):
                w2j = w2v[j, :]
                jf = jnp.full((L,), j, jnp.int32)
                jf2 = jnp.full((L,), j + HID, jnp.int32)
                for t in range(CHUNK // L):
                    av = plsc.load_gather(abuf, [erows[t], jf])
                    bv = plsc.load_gather(bbuf, [erows[t], jf2])
                    accs[t] = accs[t] + jnp.maximum(av + bv, 0.0) * w2j
            for t in range(CHUNK // L):
                off = c * CHUNK + t * L
                g = accs[t] + noise_v[pl.ds(off, L)]
                out_v[pl.ds(off, L)] = 1.0 / (1.0 + jnp.exp(-g))

        fire(0, 0)
        fire(1, 1)

        def loop_body(kk, carry):
            c0 = kk * 2
            c1 = kk * 2 + 1
            wait(c0, 0)
            compute(c0, abufs[0], bbufs[0])

            @pl.when(c0 + 2 < nch)
            def _():
                fire(c0 + 2, 0)

            wait(c1, 1)
            compute(c1, abufs[1], bbufs[1])

            @pl.when(c1 + 2 < nch)
            def _():
                fire(c1 + 2, 1)

            return carry

        lax.fori_loop(0, nch // 2, loop_body, 0)
        pltpu.sync_copy(out_v, out_hbm.at[pl.ds(base, ew)])

    return sc_kernel


def kernel(node_emb, edge_index, W1, b1, W2, b2):
    E = edge_index.shape[1]
    half = E // 2 - 1
    src = edge_index[0, :half]
    dst = edge_index[1, :half]

    bias = 0.0 + 0.0001
    eps = jax.random.uniform(
        jax.random.key(42), (half, 1),
        minval=bias, maxval=1.0 - bias, dtype=jnp.float32)
    noise = (jnp.log(eps) - jnp.log(1.0 - eps)).reshape(half) + b2[0]

    wcat = jnp.concatenate([W1[:EMB], W1[EMB:]], axis=1)
    bcat = jnp.concatenate([b1, jnp.zeros((HID,), jnp.float32)]).reshape(1, 2 * HID)
    t_tab = _precompute_tables(node_emb, wcat, bcat)

    grain = NW * CHUNK * 2
    e_pad = ((half + grain - 1) // grain) * grain
    pad = e_pad - half
    src_p = jnp.pad(src, (0, pad))
    dst_p = jnp.pad(dst, (0, pad))
    noise_p = jnp.pad(noise, (0, pad))
    w2b = W2.reshape(HID, 1) * jnp.ones((1, L), jnp.float32)

    aug_pad = _make_sc_kernel(e_pad)(t_tab, src_p, dst_p, noise_p, w2b)
    aug = aug_pad[:half]

    sym_indices = jnp.concatenate(
        [edge_index[:, :half], edge_index[::-1, :half]], axis=1)
    sym_values = jnp.concatenate([aug, aug])
    return sym_indices, sym_values, aug
